# baseline (speedup 1.0000x reference)
import functools

import jax
import jax.numpy as jnp
from jax import lax
from jax.experimental import pallas as pl
from jax.experimental.pallas import tpu as pltpu
from jax.experimental.pallas import tpu_sc as plsc

N = 10000
E = 160000
IN_DIM = 256
HEADS = 8
HID = 64
OUT_DIM = 256
NEG = 0.2

NPAD = 10240
NW = 32
EPAD = 163840
K = 128
CHA = 56
CHB = 24
CHMAX = max(CHA, CHB)
BR = 512
GRID = NPAD // BR
RPT = NPAD // 16

f32 = jnp.float32
i32 = jnp.int32


def _proj0_body(x_ref, w_ref, al_ref, ar_ref, ft_ref, el_ref, er_ref):
    f = jnp.dot(x_ref[...], w_ref[...], preferred_element_type=f32)
    for h in range(HEADS):
        fh = f[:, h * HID:(h + 1) * HID]
        ft_ref[h] = fh
        el_ref[h] = jnp.sum(fh * al_ref[h][None, :], axis=1)
        er_ref[h] = jnp.sum(fh * ar_ref[h][None, :], axis=1)


def _proj0(x, w, al, ar):
    return pl.pallas_call(
        _proj0_body,
        grid=(GRID,),
        in_specs=[
            pl.BlockSpec((BR, IN_DIM), lambda i: (i, 0)),
            pl.BlockSpec((IN_DIM, HEADS * HID), lambda i: (0, 0)),
            pl.BlockSpec((HEADS, HID), lambda i: (0, 0)),
            pl.BlockSpec((HEADS, HID), lambda i: (0, 0)),
        ],
        out_specs=[
            pl.BlockSpec((HEADS, BR, HID), lambda i: (0, i, 0)),
            pl.BlockSpec((HEADS, BR), lambda i: (0, i)),
            pl.BlockSpec((HEADS, BR), lambda i: (0, i)),
        ],
        out_shape=[
            jax.ShapeDtypeStruct((HEADS, NPAD, HID), f32),
            jax.ShapeDtypeStruct((HEADS, NPAD), f32),
            jax.ShapeDtypeStruct((HEADS, NPAD), f32),
        ],
    )(x, w, al, ar)


def _node_out(op_ref, dp_ref, b_ref):
    cols = []
    for h in range(HEADS):
        num = op_ref[0, h] + op_ref[1, h]
        den = dp_ref[0, h] + dp_ref[1, h]
        cols.append(num / (den + 1e-9)[:, None]
                    + b_ref[0, h * HID:(h + 1) * HID][None, :])
    return jnp.concatenate(cols, axis=1)


def _epi_proj_body(op_ref, dp_ref, b_ref, w_ref, al_ref, ar_ref,
                   h_ref, ft_ref, el_ref, er_ref):
    hp = jnp.maximum(_node_out(op_ref, dp_ref, b_ref), 0.0)
    h_ref[...] = hp
    f = jnp.dot(hp, w_ref[...], preferred_element_type=f32)
    for h in range(HEADS):
        fh = f[:, h * HID:(h + 1) * HID]
        ft_ref[h] = fh
        el_ref[h] = jnp.sum(fh * al_ref[h][None, :], axis=1)
        er_ref[h] = jnp.sum(fh * ar_ref[h][None, :], axis=1)


def _epi_proj(op, dp, b, w, al, ar):
    return pl.pallas_call(
        _epi_proj_body,
        grid=(GRID,),
        in_specs=[
            pl.BlockSpec((2, HEADS, BR, HID), lambda i: (0, 0, i, 0)),
            pl.BlockSpec((2, HEADS, BR), lambda i: (0, 0, i)),
            pl.BlockSpec((1, HEADS * HID), lambda i: (0, 0)),
            pl.BlockSpec((HEADS * HID, HEADS * HID), lambda i: (0, 0)),
            pl.BlockSpec((HEADS, HID), lambda i: (0, 0)),
            pl.BlockSpec((HEADS, HID), lambda i: (0, 0)),
        ],
        out_specs=[
            pl.BlockSpec((BR, HEADS * HID), lambda i: (i, 0)),
            pl.BlockSpec((HEADS, BR, HID), lambda i: (0, i, 0)),
            pl.BlockSpec((HEADS, BR), lambda i: (0, i)),
            pl.BlockSpec((HEADS, BR), lambda i: (0, i)),
        ],
        out_shape=[
            jax.ShapeDtypeStruct((NPAD, HEADS * HID), f32),
            jax.ShapeDtypeStruct((HEADS, NPAD, HID), f32),
            jax.ShapeDtypeStruct((HEADS, NPAD), f32),
            jax.ShapeDtypeStruct((HEADS, NPAD), f32),
        ],
    )(op, dp, b, w, al, ar)


def _epi_proj2_body(op_ref, dp_ref, b_ref, w_ref, al_ref, ar_ref,
                    h_ref, ft_ref, el_ref, er_ref):
    hp = jnp.maximum(_node_out(op_ref, dp_ref, b_ref), 0.0)
    h_ref[...] = hp
    f = jnp.dot(hp, w_ref[...], preferred_element_type=f32)
    for c in range(4):
        ft_ref[c] = f[:, c * 64:(c + 1) * 64]
    el_ref[0] = jnp.sum(f * al_ref[0][None, :], axis=1)
    er_ref[0] = jnp.sum(f * ar_ref[0][None, :], axis=1)


def _epi_proj2(op, dp, b, w, al, ar):
    return pl.pallas_call(
        _epi_proj2_body,
        grid=(GRID,),
        in_specs=[
            pl.BlockSpec((2, HEADS, BR, HID), lambda i: (0, 0, i, 0)),
            pl.BlockSpec((2, HEADS, BR), lambda i: (0, 0, i)),
            pl.BlockSpec((1, HEADS * HID), lambda i: (0, 0)),
            pl.BlockSpec((HEADS * HID, OUT_DIM), lambda i: (0, 0)),
            pl.BlockSpec((1, OUT_DIM), lambda i: (0, 0)),
            pl.BlockSpec((1, OUT_DIM), lambda i: (0, 0)),
        ],
        out_specs=[
            pl.BlockSpec((BR, HEADS * HID), lambda i: (i, 0)),
            pl.BlockSpec((4, BR, 64), lambda i: (0, i, 0)),
            pl.BlockSpec((1, BR), lambda i: (0, i)),
            pl.BlockSpec((1, BR), lambda i: (0, i)),
        ],
        out_shape=[
            jax.ShapeDtypeStruct((NPAD, HEADS * HID), f32),
            jax.ShapeDtypeStruct((4, NPAD, 64), f32),
            jax.ShapeDtypeStruct((1, NPAD), f32),
            jax.ShapeDtypeStruct((1, NPAD), f32),
        ],
    )(op, dp, b, w, al, ar)


def _epi2_body(op_ref, dp_ref, b_ref, h_ref):
    den = dp_ref[0] + dp_ref[1]
    for c in range(4):
        num = op_ref[0, c] + op_ref[1, c]
        h_ref[:, c * 64:(c + 1) * 64] = (
            num / (den + 1e-9)[:, None]
            + b_ref[0, c * 64:(c + 1) * 64][None, :])


def _epi2(op, dp, b):
    return pl.pallas_call(
        _epi2_body,
        grid=(GRID,),
        in_specs=[
            pl.BlockSpec((2, 4, BR, 64), lambda i: (0, 0, i, 0)),
            pl.BlockSpec((2, BR), lambda i: (0, i)),
            pl.BlockSpec((1, OUT_DIM), lambda i: (0, 0)),
        ],
        out_specs=pl.BlockSpec((BR, OUT_DIM), lambda i: (i, 0)),
        out_shape=jax.ShapeDtypeStruct((NPAD, OUT_DIM), f32),
    )(op, dp, b)


_mesh = plsc.VectorSubcoreMesh(core_axis_name="c", subcore_axis_name="s")
_sc_params = pltpu.CompilerParams(needs_layout_passes=False,
                                  use_tc_tiling_on_sc=False)


NBUF = 4


def _stage_edges(srcp, dstp, src_v, dst2d, base, chs):
    pltpu.sync_copy(srcp.at[pl.ds(base, CHMAX * K)], src_v)

    def st(i, _):
        pltpu.sync_copy(dstp.at[pl.ds(base + i * K, K)], dst2d.at[i])
        return 0
    lax.fori_loop(0, chs, st, 0)


def _zero_scratch(zrows, zden, d):
    zv = jnp.zeros((16,), f32)

    def zr(i, _):
        for j in range(d // 16):
            zrows[i, pl.ds(j * 16, 16)] = zv
        return 0
    lax.fori_loop(0, 64, zr, 0)

    def zd(i, _):
        zden[pl.ds(i * 16, 16)] = zv
        return 0
    lax.fori_loop(0, RPT // 16, zd, 0)


def _head_init(out_sh, den_sh, zrows, zden, rowbase, zsem, zero_den):
    del zsem

    def z1(i, _):
        pltpu.sync_copy(zrows, out_sh.at[pl.ds(rowbase + i * 64, 64)])
        return 0
    lax.fori_loop(0, RPT // 64, z1, 0)

    @pl.when(zero_den)
    def _():
        pltpu.sync_copy(zden, den_sh.at[pl.ds(rowbase, RPT)])


def _edge_chunks(featf, src_v, dst2d, el_v, er_v, gidx4, ee4, rows4,
                 out_sh, den_sh, gsems, ssems, esems, hoff, d, scatter_den,
                 chs):
    nvec = K // 16

    def build(ci, b):
        cb = ci * K
        for i in range(nvec):
            sv = src_v[pl.ds(cb + i * 16, 16)]
            gidx4[b, pl.ds(i * 16, 16)] = sv + hoff

    def start_gather(b):
        pltpu.async_copy(featf.at[gidx4.at[b]], rows4.at[b], gsems[b])

    def wait_gather(b):
        pltpu.make_async_copy(featf.at[gidx4.at[b]], rows4.at[b],
                              gsems[b]).wait()

    def start_scatter(ci, b):
        pltpu.async_copy(rows4.at[b], out_sh.at[dst2d.at[ci]], ssems[b],
                         add=True)

        @pl.when(scatter_den)
        def _():
            pltpu.async_copy(ee4.at[b], den_sh.at[dst2d.at[ci]], esems[b],
                             add=True)

    def wait_scatter(ci, b):
        pltpu.make_async_copy(rows4.at[b], out_sh.at[dst2d.at[ci]],
                              ssems[b]).wait()

        @pl.when(scatter_den)
        def _():
            pltpu.make_async_copy(ee4.at[b], den_sh.at[dst2d.at[ci]],
                                  esems[b]).wait()

    def compute(ci, b):
        cb = ci * K
        for i in range(nvec):
            sv = src_v[pl.ds(cb + i * 16, 16)]
            dv = dst2d[ci, pl.ds(i * 16, 16)]
            ev = plsc.load_gather(el_v, [sv])
            rv = plsc.load_gather(er_v, [dv])
            e = ev + rv
            e = jnp.where(e > 0.0, e, NEG * e)
            ee4[b, pl.ds(i * 16, 16)] = jnp.exp(e)

        @plsc.parallel_loop(0, nvec, unroll=2)
        def _(r0):
            eev = ee4[b, pl.ds(r0 * 16, 16)]
            for rr in range(16):
                r = r0 * 16 + rr
                av = jnp.full((16,), eev[rr], f32)
                for j in range(d // 16):
                    rows4[b, r, pl.ds(j * 16, 16)] = (
                        rows4[b, r, pl.ds(j * 16, 16)] * av)

    build(0, 0)
    start_gather(0)
    build(1, 1)
    start_gather(1)

    def body(cc, _):
        for u in range(NBUF):
            ci = cc * NBUF + u
            wait_gather(u)
            nb = (u + 2) % NBUF
            cn = ci + 2

            @pl.when(cn < chs)
            def _():
                @pl.when(cn >= NBUF)
                def _():
                    wait_scatter(cn - NBUF, nb)
                build(cn, nb)
                start_gather(nb)
            compute(ci, u)
            start_scatter(ci, u)
        return 0
    lax.fori_loop(0, chs // NBUF, body, 0)
    for u in range(NBUF):
        wait_scatter(chs - NBUF + u, u)


@functools.partial(
    pl.kernel, mesh=_mesh, compiler_params=_sc_params,
    out_type=[
        jax.ShapeDtypeStruct((2, HEADS, NPAD, HID), f32),
        jax.ShapeDtypeStruct((2, HEADS, NPAD), f32),
    ],
    scratch_types=[
        pltpu.VMEM((NPAD,), f32),
        pltpu.VMEM((NPAD,), f32),
        pltpu.VMEM((CHMAX * K,), i32),
        pltpu.VMEM((CHMAX, K), i32),
        pltpu.VMEM((NBUF, K), i32),
        pltpu.VMEM((NBUF, K), f32),
        pltpu.VMEM((NBUF, K, HID), f32),
        pltpu.VMEM((64, HID), f32),
        pltpu.VMEM((RPT,), f32),
        pltpu.VMEM_SHARED((NPAD, HID), f32),
        pltpu.VMEM_SHARED((NPAD,), f32),
    ] + [pltpu.SemaphoreType.DMA] * 13)
def _sc_edge8(featf, elh, erh, srcp, dstp, outp, denp,
              el_v, er_v, src_v, dst2d, gidx4, ee4, rows4, zrows, zden,
              out_sh, den_sh, g0, g1, g2, g3, s0, s1, s2, s3,
              e0, e1, e2, e3, zsem):
    gsems = (g0, g1, g2, g3)
    ssems = (s0, s1, s2, s3)
    esems = (e0, e1, e2, e3)
    c = lax.axis_index("c")
    s = lax.axis_index("s")
    chs = jnp.where(c == 0, CHA, CHB)
    base = jnp.where(c == 0, s * (CHA * K), 16 * CHA * K + s * (CHB * K))
    rowbase = s * RPT

    _stage_edges(srcp, dstp, src_v, dst2d, base, chs)
    _zero_scratch(zrows, zden, HID)

    def head_pass(h, _):
        tru = jnp.bool_(True)
        _head_init(out_sh, den_sh, zrows, zden, rowbase, zsem, tru)
        pltpu.sync_copy(elh.at[h], el_v)
        pltpu.sync_copy(erh.at[h], er_v)
        plsc.subcore_barrier()
        _edge_chunks(featf, src_v, dst2d, el_v, er_v, gidx4, ee4, rows4,
                     out_sh, den_sh, gsems, ssems, esems, h * NPAD, HID,
                     tru, chs)
        plsc.subcore_barrier()
        pltpu.sync_copy(out_sh.at[pl.ds(rowbase, RPT)],
                        outp.at[c, h, pl.ds(rowbase, RPT)])
        pltpu.sync_copy(den_sh.at[pl.ds(rowbase, RPT)],
                        denp.at[c, h, pl.ds(rowbase, RPT)])
        plsc.subcore_barrier()
        return 0
    lax.fori_loop(0, HEADS, head_pass, 0)


@functools.partial(
    pl.kernel, mesh=_mesh, compiler_params=_sc_params,
    out_type=[
        jax.ShapeDtypeStruct((2, 4, NPAD, 64), f32),
        jax.ShapeDtypeStruct((2, NPAD), f32),
    ],
    scratch_types=[
        pltpu.VMEM((NPAD,), f32),
        pltpu.VMEM((NPAD,), f32),
        pltpu.VMEM((CHMAX * K,), i32),
        pltpu.VMEM((CHMAX, K), i32),
        pltpu.VMEM((NBUF, K), i32),
        pltpu.VMEM((NBUF, K), f32),
        pltpu.VMEM((NBUF, K, 64), f32),
        pltpu.VMEM((64, 64), f32),
        pltpu.VMEM((RPT,), f32),
        pltpu.VMEM_SHARED((NPAD, 64), f32),
        pltpu.VMEM_SHARED((NPAD,), f32),
    ] + [pltpu.SemaphoreType.DMA] * 13)
def _sc_edge1(featf, elh, erh, srcp, dstp, outp, denp,
              el_v, er_v, src_v, dst2d, gidx4, ee4, rows4, zrows, zden,
              out_sh, den_sh, g0, g1, g2, g3, s0, s1, s2, s3,
              e0, e1, e2, e3, zsem):
    gsems = (g0, g1, g2, g3)
    ssems = (s0, s1, s2, s3)
    esems = (e0, e1, e2, e3)
    c = lax.axis_index("c")
    s = lax.axis_index("s")
    chs = jnp.where(c == 0, CHA, CHB)
    base = jnp.where(c == 0, s * (CHA * K), 16 * CHA * K + s * (CHB * K))
    rowbase = s * RPT

    _stage_edges(srcp, dstp, src_v, dst2d, base, chs)
    _zero_scratch(zrows, zden, 64)
    pltpu.sync_copy(elh.at[0], el_v)
    pltpu.sync_copy(erh.at[0], er_v)

    def quarter_pass(hv, _):
        zden_q = hv == 0
        _head_init(out_sh, den_sh, zrows, zden, rowbase, zsem, zden_q)
        plsc.subcore_barrier()
        _edge_chunks(featf, src_v, dst2d, el_v, er_v, gidx4, ee4, rows4,
                     out_sh, den_sh, gsems, ssems, esems, hv * NPAD, 64,
                     zden_q, chs)
        plsc.subcore_barrier()
        pltpu.sync_copy(out_sh.at[pl.ds(rowbase, RPT)],
                        outp.at[c, hv, pl.ds(rowbase, RPT)])

        @pl.when(zden_q)
        def _():
            pltpu.sync_copy(den_sh.at[pl.ds(rowbase, RPT)],
                            denp.at[c, pl.ds(rowbase, RPT)])
        plsc.subcore_barrier()
        return 0
    lax.fori_loop(0, 4, quarter_pass, 0)


@jax.jit
def kernel(feats, edge_index, W0, b0, al0, ar0, W1, b1, al1, ar1,
           W2, b2, al2, ar2):
    src = edge_index[0].astype(i32)
    dst = edge_index[1].astype(i32)
    pad = jnp.full((EPAD - E,), N, i32)
    srcp = jnp.concatenate([src, pad])
    dstp = jnp.concatenate([dst, pad])

    xp = jnp.zeros((NPAD, IN_DIM), f32).at[:N].set(feats)

    ft0, el0, er0 = _proj0(xp, W0, al0, ar0)
    op0, dp0 = _sc_edge8(ft0.reshape(HEADS * NPAD, HID), el0, er0,
                         srcp, dstp)
    h0, ft1, el1, er1 = _epi_proj(op0, dp0, b0.reshape(1, -1),
                                  W1, al1, ar1)
    op1, dp1 = _sc_edge8(ft1.reshape(HEADS * NPAD, HID), el1, er1,
                         srcp, dstp)
    h1, ft2, el2, er2 = _epi_proj2(op1, dp1, b1.reshape(1, -1),
                                   W2, al2, ar2)
    op2, dp2 = _sc_edge1(ft2.reshape(4 * NPAD, 64), el2, er2,
                         srcp, dstp)
    h2 = _epi2(op2, dp2, b2.reshape(1, -1))
    return h0[:N], h1[:N], h2[:N]


# speedup vs baseline: 1.0253x; 1.0253x over previous
import functools

import jax
import jax.numpy as jnp
from jax import lax
from jax.experimental import pallas as pl
from jax.experimental.pallas import tpu as pltpu
from jax.experimental.pallas import tpu_sc as plsc

N = 10000
E = 160000
IN_DIM = 256
HEADS = 8
HID = 64
OUT_DIM = 256
NEG = 0.2

NPAD = 10240
NW = 32
EPAD = 163840
K = 128
CHA = 56
CHB = 24
CHMAX = max(CHA, CHB)
BR = 512
GRID = NPAD // BR
RPT = NPAD // 16

f32 = jnp.float32
i32 = jnp.int32


def _proj0_body(x_ref, w_ref, al_ref, ar_ref, ft_ref, el_ref, er_ref):
    f = jnp.dot(x_ref[...], w_ref[...], preferred_element_type=f32)
    for h in range(HEADS):
        fh = f[:, h * HID:(h + 1) * HID]
        ft_ref[h] = fh
        el_ref[h] = jnp.sum(fh * al_ref[h][None, :], axis=1)
        er_ref[h] = jnp.sum(fh * ar_ref[h][None, :], axis=1)


def _proj0(x, w, al, ar):
    return pl.pallas_call(
        _proj0_body,
        grid=(GRID,),
        in_specs=[
            pl.BlockSpec((BR, IN_DIM), lambda i: (i, 0)),
            pl.BlockSpec((IN_DIM, HEADS * HID), lambda i: (0, 0)),
            pl.BlockSpec((HEADS, HID), lambda i: (0, 0)),
            pl.BlockSpec((HEADS, HID), lambda i: (0, 0)),
        ],
        out_specs=[
            pl.BlockSpec((HEADS, BR, HID), lambda i: (0, i, 0)),
            pl.BlockSpec((HEADS, BR), lambda i: (0, i)),
            pl.BlockSpec((HEADS, BR), lambda i: (0, i)),
        ],
        out_shape=[
            jax.ShapeDtypeStruct((HEADS, NPAD, HID), f32),
            jax.ShapeDtypeStruct((HEADS, NPAD), f32),
            jax.ShapeDtypeStruct((HEADS, NPAD), f32),
        ],
    )(x, w, al, ar)


def _node_out(op_ref, dp_ref, b_ref):
    cols = []
    for h in range(HEADS):
        num = op_ref[0, h] + op_ref[1, h]
        den = dp_ref[0, h] + dp_ref[1, h]
        cols.append(num / (den + 1e-9)[:, None]
                    + b_ref[0, h * HID:(h + 1) * HID][None, :])
    return jnp.concatenate(cols, axis=1)


def _epi_proj_body(op_ref, dp_ref, b_ref, w_ref, al_ref, ar_ref,
                   h_ref, ft_ref, el_ref, er_ref):
    hp = jnp.maximum(_node_out(op_ref, dp_ref, b_ref), 0.0)
    h_ref[...] = hp
    f = jnp.dot(hp, w_ref[...], preferred_element_type=f32)
    for h in range(HEADS):
        fh = f[:, h * HID:(h + 1) * HID]
        ft_ref[h] = fh
        el_ref[h] = jnp.sum(fh * al_ref[h][None, :], axis=1)
        er_ref[h] = jnp.sum(fh * ar_ref[h][None, :], axis=1)


def _epi_proj(op, dp, b, w, al, ar):
    return pl.pallas_call(
        _epi_proj_body,
        grid=(GRID,),
        in_specs=[
            pl.BlockSpec((2, HEADS, BR, HID), lambda i: (0, 0, i, 0)),
            pl.BlockSpec((2, HEADS, BR), lambda i: (0, 0, i)),
            pl.BlockSpec((1, HEADS * HID), lambda i: (0, 0)),
            pl.BlockSpec((HEADS * HID, HEADS * HID), lambda i: (0, 0)),
            pl.BlockSpec((HEADS, HID), lambda i: (0, 0)),
            pl.BlockSpec((HEADS, HID), lambda i: (0, 0)),
        ],
        out_specs=[
            pl.BlockSpec((BR, HEADS * HID), lambda i: (i, 0)),
            pl.BlockSpec((HEADS, BR, HID), lambda i: (0, i, 0)),
            pl.BlockSpec((HEADS, BR), lambda i: (0, i)),
            pl.BlockSpec((HEADS, BR), lambda i: (0, i)),
        ],
        out_shape=[
            jax.ShapeDtypeStruct((NPAD, HEADS * HID), f32),
            jax.ShapeDtypeStruct((HEADS, NPAD, HID), f32),
            jax.ShapeDtypeStruct((HEADS, NPAD), f32),
            jax.ShapeDtypeStruct((HEADS, NPAD), f32),
        ],
    )(op, dp, b, w, al, ar)


def _epi_proj2_body(op_ref, dp_ref, b_ref, w_ref, al_ref, ar_ref,
                    h_ref, ft_ref, el_ref, er_ref):
    hp = jnp.maximum(_node_out(op_ref, dp_ref, b_ref), 0.0)
    h_ref[...] = hp
    f = jnp.dot(hp, w_ref[...], preferred_element_type=f32)
    for c in range(4):
        ft_ref[c] = f[:, c * 64:(c + 1) * 64]
    el_ref[0] = jnp.sum(f * al_ref[0][None, :], axis=1)
    er_ref[0] = jnp.sum(f * ar_ref[0][None, :], axis=1)


def _epi_proj2(op, dp, b, w, al, ar):
    return pl.pallas_call(
        _epi_proj2_body,
        grid=(GRID,),
        in_specs=[
            pl.BlockSpec((2, HEADS, BR, HID), lambda i: (0, 0, i, 0)),
            pl.BlockSpec((2, HEADS, BR), lambda i: (0, 0, i)),
            pl.BlockSpec((1, HEADS * HID), lambda i: (0, 0)),
            pl.BlockSpec((HEADS * HID, OUT_DIM), lambda i: (0, 0)),
            pl.BlockSpec((1, OUT_DIM), lambda i: (0, 0)),
            pl.BlockSpec((1, OUT_DIM), lambda i: (0, 0)),
        ],
        out_specs=[
            pl.BlockSpec((BR, HEADS * HID), lambda i: (i, 0)),
            pl.BlockSpec((4, BR, 64), lambda i: (0, i, 0)),
            pl.BlockSpec((1, BR), lambda i: (0, i)),
            pl.BlockSpec((1, BR), lambda i: (0, i)),
        ],
        out_shape=[
            jax.ShapeDtypeStruct((NPAD, HEADS * HID), f32),
            jax.ShapeDtypeStruct((4, NPAD, 64), f32),
            jax.ShapeDtypeStruct((1, NPAD), f32),
            jax.ShapeDtypeStruct((1, NPAD), f32),
        ],
    )(op, dp, b, w, al, ar)


def _epi2_body(op_ref, dp_ref, b_ref, h_ref):
    den = dp_ref[0] + dp_ref[1]
    for c in range(4):
        num = op_ref[0, c] + op_ref[1, c]
        h_ref[:, c * 64:(c + 1) * 64] = (
            num / (den + 1e-9)[:, None]
            + b_ref[0, c * 64:(c + 1) * 64][None, :])


def _epi2(op, dp, b):
    return pl.pallas_call(
        _epi2_body,
        grid=(GRID,),
        in_specs=[
            pl.BlockSpec((2, 4, BR, 64), lambda i: (0, 0, i, 0)),
            pl.BlockSpec((2, BR), lambda i: (0, i)),
            pl.BlockSpec((1, OUT_DIM), lambda i: (0, 0)),
        ],
        out_specs=pl.BlockSpec((BR, OUT_DIM), lambda i: (i, 0)),
        out_shape=jax.ShapeDtypeStruct((NPAD, OUT_DIM), f32),
    )(op, dp, b)


_mesh = plsc.VectorSubcoreMesh(core_axis_name="c", subcore_axis_name="s")
_sc_params = pltpu.CompilerParams(needs_layout_passes=False,
                                  use_tc_tiling_on_sc=False)


NBUF = 4


def _stage_edges(srcp, dstp, src_v, dst2d, base, chs, zsem):
    pltpu.async_copy(srcp.at[pl.ds(base, CHMAX * K)], src_v, zsem)

    def group(g, _):
        for j in range(8):
            i = g * 8 + j
            pltpu.async_copy(dstp.at[pl.ds(base + i * K, K)],
                             dst2d.at[i], zsem)
        for j in range(8):
            i = g * 8 + j
            pltpu.make_async_copy(dstp.at[pl.ds(base + i * K, K)],
                                  dst2d.at[i], zsem).wait()
        return 0
    lax.fori_loop(0, chs // 8, group, 0)
    pltpu.make_async_copy(srcp.at[pl.ds(base, CHMAX * K)], src_v,
                          zsem).wait()


def _zero_scratch(zrows, zden, d):
    zv = jnp.zeros((16,), f32)

    def zr(i, _):
        for j in range(d // 16):
            zrows[i, pl.ds(j * 16, 16)] = zv
        return 0
    lax.fori_loop(0, 64, zr, 0)

    def zd(i, _):
        zden[pl.ds(i * 16, 16)] = zv
        return 0
    lax.fori_loop(0, RPT // 16, zd, 0)


def _head_init(out_sh, den_sh, zrows, zden, rowbase, zsem, zero_den):
    for half in range(2):
        for i in range(5):
            pltpu.async_copy(
                zrows, out_sh.at[pl.ds(rowbase + (half * 5 + i) * 64, 64)],
                zsem)
        for i in range(5):
            pltpu.make_async_copy(
                zrows, out_sh.at[pl.ds(rowbase + (half * 5 + i) * 64, 64)],
                zsem).wait()

    @pl.when(zero_den)
    def _():
        pltpu.sync_copy(zden, den_sh.at[pl.ds(rowbase, RPT)])


def _edge_chunks(featf, src_v, dst2d, el_v, er_v, gidx4, ee4, rows4,
                 out_sh, den_sh, gsems, ssems, esems, hoff, d, scatter_den,
                 chs):
    nvec = K // 16

    def build(ci, b):
        cb = ci * K
        for i in range(nvec):
            sv = src_v[pl.ds(cb + i * 16, 16)]
            gidx4[b, pl.ds(i * 16, 16)] = sv + hoff

    def start_gather(b):
        pltpu.async_copy(featf.at[gidx4.at[b]], rows4.at[b], gsems[b])

    def wait_gather(b):
        pltpu.make_async_copy(featf.at[gidx4.at[b]], rows4.at[b],
                              gsems[b]).wait()

    def start_scatter(ci, b):
        pltpu.async_copy(rows4.at[b], out_sh.at[dst2d.at[ci]], ssems[b],
                         add=True)

        @pl.when(scatter_den)
        def _():
            pltpu.async_copy(ee4.at[b], den_sh.at[dst2d.at[ci]], esems[b],
                             add=True)

    def wait_scatter(ci, b):
        pltpu.make_async_copy(rows4.at[b], out_sh.at[dst2d.at[ci]],
                              ssems[b]).wait()

        @pl.when(scatter_den)
        def _():
            pltpu.make_async_copy(ee4.at[b], den_sh.at[dst2d.at[ci]],
                                  esems[b]).wait()

    def compute(ci, b):
        cb = ci * K
        for i in range(nvec):
            sv = src_v[pl.ds(cb + i * 16, 16)]
            dv = dst2d[ci, pl.ds(i * 16, 16)]
            ev = plsc.load_gather(el_v, [sv])
            rv = plsc.load_gather(er_v, [dv])
            e = ev + rv
            e = jnp.where(e > 0.0, e, NEG * e)
            ee4[b, pl.ds(i * 16, 16)] = jnp.exp(e)

        def scale(r0, _):
            eev = ee4[b, pl.ds(r0 * 16, 16)]
            for rr in range(16):
                r = r0 * 16 + rr
                av = jnp.full((16,), eev[rr], f32)
                for j in range(d // 16):
                    rows4[b, r, pl.ds(j * 16, 16)] = (
                        rows4[b, r, pl.ds(j * 16, 16)] * av)
            return 0
        lax.fori_loop(0, nvec, scale, 0)

    build(0, 0)
    start_gather(0)
    build(1, 1)
    start_gather(1)

    def body(cc, _):
        for u in range(NBUF):
            ci = cc * NBUF + u
            wait_gather(u)
            nb = (u + 2) % NBUF
            cn = ci + 2

            @pl.when(cn < chs)
            def _():
                @pl.when(cn >= NBUF)
                def _():
                    wait_scatter(cn - NBUF, nb)
                build(cn, nb)
                start_gather(nb)
            compute(ci, u)
            start_scatter(ci, u)
        return 0
    lax.fori_loop(0, chs // NBUF, body, 0)
    for u in range(NBUF):
        wait_scatter(chs - NBUF + u, u)


@functools.partial(
    pl.kernel, mesh=_mesh, compiler_params=_sc_params,
    out_type=[
        jax.ShapeDtypeStruct((2, HEADS, NPAD, HID), f32),
        jax.ShapeDtypeStruct((2, HEADS, NPAD), f32),
    ],
    scratch_types=[
        pltpu.VMEM((NPAD,), f32),
        pltpu.VMEM((NPAD,), f32),
        pltpu.VMEM((CHMAX * K,), i32),
        pltpu.VMEM((CHMAX, K), i32),
        pltpu.VMEM((NBUF, K), i32),
        pltpu.VMEM((NBUF, K), f32),
        pltpu.VMEM((NBUF, K, HID), f32),
        pltpu.VMEM((64, HID), f32),
        pltpu.VMEM((RPT,), f32),
        pltpu.VMEM_SHARED((NPAD, HID), f32),
        pltpu.VMEM_SHARED((NPAD,), f32),
    ] + [pltpu.SemaphoreType.DMA] * 13)
def _sc_edge8(featf, elh, erh, srcp, dstp, outp, denp,
              el_v, er_v, src_v, dst2d, gidx4, ee4, rows4, zrows, zden,
              out_sh, den_sh, g0, g1, g2, g3, s0, s1, s2, s3,
              e0, e1, e2, e3, zsem):
    gsems = (g0, g1, g2, g3)
    ssems = (s0, s1, s2, s3)
    esems = (e0, e1, e2, e3)
    c = lax.axis_index("c")
    s = lax.axis_index("s")
    chs = jnp.where(c == 0, CHA, CHB)
    base = jnp.where(c == 0, s * (CHA * K), 16 * CHA * K + s * (CHB * K))
    rowbase = s * RPT

    _stage_edges(srcp, dstp, src_v, dst2d, base, chs, zsem)
    _zero_scratch(zrows, zden, HID)

    def head_pass(h, _):
        tru = jnp.bool_(True)
        _head_init(out_sh, den_sh, zrows, zden, rowbase, zsem, tru)
        pltpu.sync_copy(elh.at[h], el_v)
        pltpu.sync_copy(erh.at[h], er_v)
        plsc.subcore_barrier()
        _edge_chunks(featf, src_v, dst2d, el_v, er_v, gidx4, ee4, rows4,
                     out_sh, den_sh, gsems, ssems, esems, h * NPAD, HID,
                     tru, chs)
        plsc.subcore_barrier()
        pltpu.sync_copy(out_sh.at[pl.ds(rowbase, RPT)],
                        outp.at[c, h, pl.ds(rowbase, RPT)])
        pltpu.sync_copy(den_sh.at[pl.ds(rowbase, RPT)],
                        denp.at[c, h, pl.ds(rowbase, RPT)])
        plsc.subcore_barrier()
        return 0
    lax.fori_loop(0, HEADS, head_pass, 0)


@functools.partial(
    pl.kernel, mesh=_mesh, compiler_params=_sc_params,
    out_type=[
        jax.ShapeDtypeStruct((2, 4, NPAD, 64), f32),
        jax.ShapeDtypeStruct((2, NPAD), f32),
    ],
    scratch_types=[
        pltpu.VMEM((NPAD,), f32),
        pltpu.VMEM((NPAD,), f32),
        pltpu.VMEM((CHMAX * K,), i32),
        pltpu.VMEM((CHMAX, K), i32),
        pltpu.VMEM((NBUF, K), i32),
        pltpu.VMEM((NBUF, K), f32),
        pltpu.VMEM((NBUF, K, 64), f32),
        pltpu.VMEM((64, 64), f32),
        pltpu.VMEM((RPT,), f32),
        pltpu.VMEM_SHARED((NPAD, 64), f32),
        pltpu.VMEM_SHARED((NPAD,), f32),
    ] + [pltpu.SemaphoreType.DMA] * 13)
def _sc_edge1(featf, elh, erh, srcp, dstp, outp, denp,
              el_v, er_v, src_v, dst2d, gidx4, ee4, rows4, zrows, zden,
              out_sh, den_sh, g0, g1, g2, g3, s0, s1, s2, s3,
              e0, e1, e2, e3, zsem):
    gsems = (g0, g1, g2, g3)
    ssems = (s0, s1, s2, s3)
    esems = (e0, e1, e2, e3)
    c = lax.axis_index("c")
    s = lax.axis_index("s")
    chs = jnp.where(c == 0, CHA, CHB)
    base = jnp.where(c == 0, s * (CHA * K), 16 * CHA * K + s * (CHB * K))
    rowbase = s * RPT

    _stage_edges(srcp, dstp, src_v, dst2d, base, chs, zsem)
    _zero_scratch(zrows, zden, 64)
    pltpu.sync_copy(elh.at[0], el_v)
    pltpu.sync_copy(erh.at[0], er_v)

    def quarter_pass(hv, _):
        zden_q = hv == 0
        _head_init(out_sh, den_sh, zrows, zden, rowbase, zsem, zden_q)
        plsc.subcore_barrier()
        _edge_chunks(featf, src_v, dst2d, el_v, er_v, gidx4, ee4, rows4,
                     out_sh, den_sh, gsems, ssems, esems, hv * NPAD, 64,
                     zden_q, chs)
        plsc.subcore_barrier()
        pltpu.sync_copy(out_sh.at[pl.ds(rowbase, RPT)],
                        outp.at[c, hv, pl.ds(rowbase, RPT)])

        @pl.when(zden_q)
        def _():
            pltpu.sync_copy(den_sh.at[pl.ds(rowbase, RPT)],
                            denp.at[c, pl.ds(rowbase, RPT)])
        plsc.subcore_barrier()
        return 0
    lax.fori_loop(0, 4, quarter_pass, 0)


@jax.jit
def kernel(feats, edge_index, W0, b0, al0, ar0, W1, b1, al1, ar1,
           W2, b2, al2, ar2):
    src = edge_index[0].astype(i32)
    dst = edge_index[1].astype(i32)
    pad = jnp.full((EPAD - E,), N, i32)
    srcp = jnp.concatenate([src, pad])
    dstp = jnp.concatenate([dst, pad])

    xp = jnp.zeros((NPAD, IN_DIM), f32).at[:N].set(feats)

    ft0, el0, er0 = _proj0(xp, W0, al0, ar0)
    op0, dp0 = _sc_edge8(ft0.reshape(HEADS * NPAD, HID), el0, er0,
                         srcp, dstp)
    h0, ft1, el1, er1 = _epi_proj(op0, dp0, b0.reshape(1, -1),
                                  W1, al1, ar1)
    op1, dp1 = _sc_edge8(ft1.reshape(HEADS * NPAD, HID), el1, er1,
                         srcp, dstp)
    h1, ft2, el2, er2 = _epi_proj2(op1, dp1, b1.reshape(1, -1),
                                   W2, al2, ar2)
    op2, dp2 = _sc_edge1(ft2.reshape(4 * NPAD, 64), el2, er2,
                         srcp, dstp)
    h2 = _epi2(op2, dp2, b2.reshape(1, -1))
    return h0[:N], h1[:N], h2[:N]


# speedup vs baseline: 1.0313x; 1.0059x over previous
import functools

import jax
import jax.numpy as jnp
from jax import lax
from jax.experimental import pallas as pl
from jax.experimental.pallas import tpu as pltpu
from jax.experimental.pallas import tpu_sc as plsc

N = 10000
E = 160000
IN_DIM = 256
HEADS = 8
HID = 64
OUT_DIM = 256
NEG = 0.2

NPAD = 10240
NW = 32
EPAD = 163840
K = 128
CHA = 56
CHB = 24
CHMAX = max(CHA, CHB)
BR = 1024
GRID = NPAD // BR
RPT = NPAD // 16

f32 = jnp.float32
i32 = jnp.int32


def _proj0_body(x_ref, w_ref, al_ref, ar_ref, ft_ref, el_ref, er_ref):
    f = jnp.dot(x_ref[...], w_ref[...], preferred_element_type=f32)
    for h in range(HEADS):
        fh = f[:, h * HID:(h + 1) * HID]
        ft_ref[h] = fh
        el_ref[h] = jnp.sum(fh * al_ref[h][None, :], axis=1)
        er_ref[h] = jnp.sum(fh * ar_ref[h][None, :], axis=1)


def _proj0(x, w, al, ar):
    return pl.pallas_call(
        _proj0_body,
        grid=(GRID,),
        in_specs=[
            pl.BlockSpec((BR, IN_DIM), lambda i: (i, 0)),
            pl.BlockSpec((IN_DIM, HEADS * HID), lambda i: (0, 0)),
            pl.BlockSpec((HEADS, HID), lambda i: (0, 0)),
            pl.BlockSpec((HEADS, HID), lambda i: (0, 0)),
        ],
        out_specs=[
            pl.BlockSpec((HEADS, BR, HID), lambda i: (0, i, 0)),
            pl.BlockSpec((HEADS, BR), lambda i: (0, i)),
            pl.BlockSpec((HEADS, BR), lambda i: (0, i)),
        ],
        out_shape=[
            jax.ShapeDtypeStruct((HEADS, NPAD, HID), f32),
            jax.ShapeDtypeStruct((HEADS, NPAD), f32),
            jax.ShapeDtypeStruct((HEADS, NPAD), f32),
        ],
    )(x, w, al, ar)


def _node_out(op_ref, dp_ref, b_ref):
    cols = []
    for h in range(HEADS):
        num = op_ref[0, h] + op_ref[1, h]
        den = dp_ref[0, h] + dp_ref[1, h]
        cols.append(num / (den + 1e-9)[:, None]
                    + b_ref[0, h * HID:(h + 1) * HID][None, :])
    return jnp.concatenate(cols, axis=1)


def _epi_proj_body(op_ref, dp_ref, b_ref, w_ref, al_ref, ar_ref,
                   h_ref, ft_ref, el_ref, er_ref):
    hp = jnp.maximum(_node_out(op_ref, dp_ref, b_ref), 0.0)
    h_ref[...] = hp
    f = jnp.dot(hp, w_ref[...], preferred_element_type=f32)
    for h in range(HEADS):
        fh = f[:, h * HID:(h + 1) * HID]
        ft_ref[h] = fh
        el_ref[h] = jnp.sum(fh * al_ref[h][None, :], axis=1)
        er_ref[h] = jnp.sum(fh * ar_ref[h][None, :], axis=1)


def _epi_proj(op, dp, b, w, al, ar):
    return pl.pallas_call(
        _epi_proj_body,
        grid=(GRID,),
        in_specs=[
            pl.BlockSpec((2, HEADS, BR, HID), lambda i: (0, 0, i, 0)),
            pl.BlockSpec((2, HEADS, BR), lambda i: (0, 0, i)),
            pl.BlockSpec((1, HEADS * HID), lambda i: (0, 0)),
            pl.BlockSpec((HEADS * HID, HEADS * HID), lambda i: (0, 0)),
            pl.BlockSpec((HEADS, HID), lambda i: (0, 0)),
            pl.BlockSpec((HEADS, HID), lambda i: (0, 0)),
        ],
        out_specs=[
            pl.BlockSpec((BR, HEADS * HID), lambda i: (i, 0)),
            pl.BlockSpec((HEADS, BR, HID), lambda i: (0, i, 0)),
            pl.BlockSpec((HEADS, BR), lambda i: (0, i)),
            pl.BlockSpec((HEADS, BR), lambda i: (0, i)),
        ],
        out_shape=[
            jax.ShapeDtypeStruct((NPAD, HEADS * HID), f32),
            jax.ShapeDtypeStruct((HEADS, NPAD, HID), f32),
            jax.ShapeDtypeStruct((HEADS, NPAD), f32),
            jax.ShapeDtypeStruct((HEADS, NPAD), f32),
        ],
    )(op, dp, b, w, al, ar)


def _epi_proj2_body(op_ref, dp_ref, b_ref, w_ref, al_ref, ar_ref,
                    h_ref, ft_ref, el_ref, er_ref):
    hp = jnp.maximum(_node_out(op_ref, dp_ref, b_ref), 0.0)
    h_ref[...] = hp
    f = jnp.dot(hp, w_ref[...], preferred_element_type=f32)
    for c in range(4):
        ft_ref[c] = f[:, c * 64:(c + 1) * 64]
    el_ref[0] = jnp.sum(f * al_ref[0][None, :], axis=1)
    er_ref[0] = jnp.sum(f * ar_ref[0][None, :], axis=1)


def _epi_proj2(op, dp, b, w, al, ar):
    return pl.pallas_call(
        _epi_proj2_body,
        grid=(GRID,),
        in_specs=[
            pl.BlockSpec((2, HEADS, BR, HID), lambda i: (0, 0, i, 0)),
            pl.BlockSpec((2, HEADS, BR), lambda i: (0, 0, i)),
            pl.BlockSpec((1, HEADS * HID), lambda i: (0, 0)),
            pl.BlockSpec((HEADS * HID, OUT_DIM), lambda i: (0, 0)),
            pl.BlockSpec((1, OUT_DIM), lambda i: (0, 0)),
            pl.BlockSpec((1, OUT_DIM), lambda i: (0, 0)),
        ],
        out_specs=[
            pl.BlockSpec((BR, HEADS * HID), lambda i: (i, 0)),
            pl.BlockSpec((4, BR, 64), lambda i: (0, i, 0)),
            pl.BlockSpec((1, BR), lambda i: (0, i)),
            pl.BlockSpec((1, BR), lambda i: (0, i)),
        ],
        out_shape=[
            jax.ShapeDtypeStruct((NPAD, HEADS * HID), f32),
            jax.ShapeDtypeStruct((4, NPAD, 64), f32),
            jax.ShapeDtypeStruct((1, NPAD), f32),
            jax.ShapeDtypeStruct((1, NPAD), f32),
        ],
    )(op, dp, b, w, al, ar)


def _epi2_body(op_ref, dp_ref, b_ref, h_ref):
    den = dp_ref[0] + dp_ref[1]
    for c in range(4):
        num = op_ref[0, c] + op_ref[1, c]
        h_ref[:, c * 64:(c + 1) * 64] = (
            num / (den + 1e-9)[:, None]
            + b_ref[0, c * 64:(c + 1) * 64][None, :])


def _epi2(op, dp, b):
    return pl.pallas_call(
        _epi2_body,
        grid=(GRID,),
        in_specs=[
            pl.BlockSpec((2, 4, BR, 64), lambda i: (0, 0, i, 0)),
            pl.BlockSpec((2, BR), lambda i: (0, i)),
            pl.BlockSpec((1, OUT_DIM), lambda i: (0, 0)),
        ],
        out_specs=pl.BlockSpec((BR, OUT_DIM), lambda i: (i, 0)),
        out_shape=jax.ShapeDtypeStruct((NPAD, OUT_DIM), f32),
    )(op, dp, b)


_mesh = plsc.VectorSubcoreMesh(core_axis_name="c", subcore_axis_name="s")
_sc_params = pltpu.CompilerParams(needs_layout_passes=False,
                                  use_tc_tiling_on_sc=False)


NBUF = 4


def _stage_edges(srcp, dstp, src_v, dst2d, base, chs, zsem):
    pltpu.async_copy(srcp.at[pl.ds(base, CHMAX * K)], src_v, zsem)

    def group(g, _):
        for j in range(8):
            i = g * 8 + j
            pltpu.async_copy(dstp.at[pl.ds(base + i * K, K)],
                             dst2d.at[i], zsem)
        for j in range(8):
            i = g * 8 + j
            pltpu.make_async_copy(dstp.at[pl.ds(base + i * K, K)],
                                  dst2d.at[i], zsem).wait()
        return 0
    lax.fori_loop(0, chs // 8, group, 0)
    pltpu.make_async_copy(srcp.at[pl.ds(base, CHMAX * K)], src_v,
                          zsem).wait()


def _zero_scratch(zrows, zden, d):
    zv = jnp.zeros((16,), f32)

    def zr(i, _):
        for j in range(d // 16):
            zrows[i, pl.ds(j * 16, 16)] = zv
        return 0
    lax.fori_loop(0, 64, zr, 0)

    def zd(i, _):
        zden[pl.ds(i * 16, 16)] = zv
        return 0
    lax.fori_loop(0, RPT // 16, zd, 0)


def _head_init(out_sh, den_sh, zrows, zden, rowbase, zsem, zero_den):
    for half in range(2):
        for i in range(5):
            pltpu.async_copy(
                zrows, out_sh.at[pl.ds(rowbase + (half * 5 + i) * 64, 64)],
                zsem)
        for i in range(5):
            pltpu.make_async_copy(
                zrows, out_sh.at[pl.ds(rowbase + (half * 5 + i) * 64, 64)],
                zsem).wait()

    @pl.when(zero_den)
    def _():
        pltpu.sync_copy(zden, den_sh.at[pl.ds(rowbase, RPT)])


def _edge_chunks(featf, src_v, dst2d, el_v, er_v, gidx4, ee4, rows4,
                 out_sh, den_sh, gsems, ssems, esems, hoff, d, scatter_den,
                 chs):
    nvec = K // 16

    def build(ci, b):
        cb = ci * K
        for i in range(nvec):
            sv = src_v[pl.ds(cb + i * 16, 16)]
            gidx4[b, pl.ds(i * 16, 16)] = sv + hoff

    def start_gather(b):
        pltpu.async_copy(featf.at[gidx4.at[b]], rows4.at[b], gsems[b])

    def wait_gather(b):
        pltpu.make_async_copy(featf.at[gidx4.at[b]], rows4.at[b],
                              gsems[b]).wait()

    def start_scatter(ci, b):
        pltpu.async_copy(rows4.at[b], out_sh.at[dst2d.at[ci]], ssems[b],
                         add=True)

        @pl.when(scatter_den)
        def _():
            pltpu.async_copy(ee4.at[b], den_sh.at[dst2d.at[ci]], esems[b],
                             add=True)

    def wait_scatter(ci, b):
        pltpu.make_async_copy(rows4.at[b], out_sh.at[dst2d.at[ci]],
                              ssems[b]).wait()

        @pl.when(scatter_den)
        def _():
            pltpu.make_async_copy(ee4.at[b], den_sh.at[dst2d.at[ci]],
                                  esems[b]).wait()

    def compute(ci, b):
        cb = ci * K
        for i in range(nvec):
            sv = src_v[pl.ds(cb + i * 16, 16)]
            dv = dst2d[ci, pl.ds(i * 16, 16)]
            ev = plsc.load_gather(el_v, [sv])
            rv = plsc.load_gather(er_v, [dv])
            e = ev + rv
            e = jnp.where(e > 0.0, e, NEG * e)
            ee4[b, pl.ds(i * 16, 16)] = jnp.exp(e)

        def scale(r0, _):
            eev = ee4[b, pl.ds(r0 * 16, 16)]
            for rr in range(16):
                r = r0 * 16 + rr
                av = jnp.full((16,), eev[rr], f32)
                for j in range(d // 16):
                    rows4[b, r, pl.ds(j * 16, 16)] = (
                        rows4[b, r, pl.ds(j * 16, 16)] * av)
            return 0
        lax.fori_loop(0, nvec, scale, 0)

    build(0, 0)
    start_gather(0)
    build(1, 1)
    start_gather(1)

    def body(cc, _):
        for u in range(NBUF):
            ci = cc * NBUF + u
            wait_gather(u)
            nb = (u + 2) % NBUF
            cn = ci + 2

            @pl.when(cn < chs)
            def _():
                @pl.when(cn >= NBUF)
                def _():
                    wait_scatter(cn - NBUF, nb)
                build(cn, nb)
                start_gather(nb)
            compute(ci, u)
            start_scatter(ci, u)
        return 0
    lax.fori_loop(0, chs // NBUF, body, 0)
    for u in range(NBUF):
        wait_scatter(chs - NBUF + u, u)


@functools.partial(
    pl.kernel, mesh=_mesh, compiler_params=_sc_params,
    out_type=[
        jax.ShapeDtypeStruct((2, HEADS, NPAD, HID), f32),
        jax.ShapeDtypeStruct((2, HEADS, NPAD), f32),
    ],
    scratch_types=[
        pltpu.VMEM((NPAD,), f32),
        pltpu.VMEM((NPAD,), f32),
        pltpu.VMEM((CHMAX * K,), i32),
        pltpu.VMEM((CHMAX, K), i32),
        pltpu.VMEM((NBUF, K), i32),
        pltpu.VMEM((NBUF, K), f32),
        pltpu.VMEM((NBUF, K, HID), f32),
        pltpu.VMEM((64, HID), f32),
        pltpu.VMEM((RPT,), f32),
        pltpu.VMEM_SHARED((NPAD, HID), f32),
        pltpu.VMEM_SHARED((NPAD,), f32),
    ] + [pltpu.SemaphoreType.DMA] * 13)
def _sc_edge8(featf, elh, erh, srcp, dstp, outp, denp,
              el_v, er_v, src_v, dst2d, gidx4, ee4, rows4, zrows, zden,
              out_sh, den_sh, g0, g1, g2, g3, s0, s1, s2, s3,
              e0, e1, e2, e3, zsem):
    gsems = (g0, g1, g2, g3)
    ssems = (s0, s1, s2, s3)
    esems = (e0, e1, e2, e3)
    c = lax.axis_index("c")
    s = lax.axis_index("s")
    chs = jnp.where(c == 0, CHA, CHB)
    base = jnp.where(c == 0, s * (CHA * K), 16 * CHA * K + s * (CHB * K))
    rowbase = s * RPT

    _stage_edges(srcp, dstp, src_v, dst2d, base, chs, zsem)
    _zero_scratch(zrows, zden, HID)

    def head_pass(h, _):
        tru = jnp.bool_(True)
        _head_init(out_sh, den_sh, zrows, zden, rowbase, zsem, tru)
        pltpu.sync_copy(elh.at[h], el_v)
        pltpu.sync_copy(erh.at[h], er_v)
        plsc.subcore_barrier()
        _edge_chunks(featf, src_v, dst2d, el_v, er_v, gidx4, ee4, rows4,
                     out_sh, den_sh, gsems, ssems, esems, h * NPAD, HID,
                     tru, chs)
        plsc.subcore_barrier()
        pltpu.sync_copy(out_sh.at[pl.ds(rowbase, RPT)],
                        outp.at[c, h, pl.ds(rowbase, RPT)])
        pltpu.sync_copy(den_sh.at[pl.ds(rowbase, RPT)],
                        denp.at[c, h, pl.ds(rowbase, RPT)])
        plsc.subcore_barrier()
        return 0
    lax.fori_loop(0, HEADS, head_pass, 0)


@functools.partial(
    pl.kernel, mesh=_mesh, compiler_params=_sc_params,
    out_type=[
        jax.ShapeDtypeStruct((2, 4, NPAD, 64), f32),
        jax.ShapeDtypeStruct((2, NPAD), f32),
    ],
    scratch_types=[
        pltpu.VMEM((NPAD,), f32),
        pltpu.VMEM((NPAD,), f32),
        pltpu.VMEM((CHMAX * K,), i32),
        pltpu.VMEM((CHMAX, K), i32),
        pltpu.VMEM((NBUF, K), i32),
        pltpu.VMEM((NBUF, K), f32),
        pltpu.VMEM((NBUF, K, 64), f32),
        pltpu.VMEM((64, 64), f32),
        pltpu.VMEM((RPT,), f32),
        pltpu.VMEM_SHARED((NPAD, 64), f32),
        pltpu.VMEM_SHARED((NPAD,), f32),
    ] + [pltpu.SemaphoreType.DMA] * 13)
def _sc_edge1(featf, elh, erh, srcp, dstp, outp, denp,
              el_v, er_v, src_v, dst2d, gidx4, ee4, rows4, zrows, zden,
              out_sh, den_sh, g0, g1, g2, g3, s0, s1, s2, s3,
              e0, e1, e2, e3, zsem):
    gsems = (g0, g1, g2, g3)
    ssems = (s0, s1, s2, s3)
    esems = (e0, e1, e2, e3)
    c = lax.axis_index("c")
    s = lax.axis_index("s")
    chs = jnp.where(c == 0, CHA, CHB)
    base = jnp.where(c == 0, s * (CHA * K), 16 * CHA * K + s * (CHB * K))
    rowbase = s * RPT

    _stage_edges(srcp, dstp, src_v, dst2d, base, chs, zsem)
    _zero_scratch(zrows, zden, 64)
    pltpu.sync_copy(elh.at[0], el_v)
    pltpu.sync_copy(erh.at[0], er_v)

    def quarter_pass(hv, _):
        zden_q = hv == 0
        _head_init(out_sh, den_sh, zrows, zden, rowbase, zsem, zden_q)
        plsc.subcore_barrier()
        _edge_chunks(featf, src_v, dst2d, el_v, er_v, gidx4, ee4, rows4,
                     out_sh, den_sh, gsems, ssems, esems, hv * NPAD, 64,
                     zden_q, chs)
        plsc.subcore_barrier()
        pltpu.sync_copy(out_sh.at[pl.ds(rowbase, RPT)],
                        outp.at[c, hv, pl.ds(rowbase, RPT)])

        @pl.when(zden_q)
        def _():
            pltpu.sync_copy(den_sh.at[pl.ds(rowbase, RPT)],
                            denp.at[c, pl.ds(rowbase, RPT)])
        plsc.subcore_barrier()
        return 0
    lax.fori_loop(0, 4, quarter_pass, 0)


@jax.jit
def kernel(feats, edge_index, W0, b0, al0, ar0, W1, b1, al1, ar1,
           W2, b2, al2, ar2):
    src = edge_index[0].astype(i32)
    dst = edge_index[1].astype(i32)
    pad = jnp.full((EPAD - E,), N, i32)
    srcp = jnp.concatenate([src, pad])
    dstp = jnp.concatenate([dst, pad])

    xp = jnp.zeros((NPAD, IN_DIM), f32).at[:N].set(feats)

    ft0, el0, er0 = _proj0(xp, W0, al0, ar0)
    op0, dp0 = _sc_edge8(ft0.reshape(HEADS * NPAD, HID), el0, er0,
                         srcp, dstp)
    h0, ft1, el1, er1 = _epi_proj(op0, dp0, b0.reshape(1, -1),
                                  W1, al1, ar1)
    op1, dp1 = _sc_edge8(ft1.reshape(HEADS * NPAD, HID), el1, er1,
                         srcp, dstp)
    h1, ft2, el2, er2 = _epi_proj2(op1, dp1, b1.reshape(1, -1),
                                   W2, al2, ar2)
    op2, dp2 = _sc_edge1(ft2.reshape(4 * NPAD, 64), el2, er2,
                         srcp, dstp)
    h2 = _epi2(op2, dp2, b2.reshape(1, -1))
    return h0[:N], h1[:N], h2[:N]
